# baseline (speedup 1.0000x reference)
import functools

import jax
import jax.numpy as jnp
from jax import lax
from jax.experimental import pallas as pl
from jax.experimental.pallas import tpu as pltpu
from jax.experimental.pallas import tpu_sc as plsc

N = 10000
NRELS = 18
RTOT = 2 * NRELS + 1
D = 128
NE = 320000
E2 = 2 * NE
NT = 30000

NC, NS = 2, 16
NW = NC * NS
EPW = E2 // NW
KB = 40
NBLK = EPW // KB
RPT = N // NS
ZROWS = 25
CHB = 100
RING = 4

NTP = 30720
KB2 = 120
NBLK2 = (NTP // NW) // KB2


def _transform1(emb, bias, W):
    def body(e_ref, b_ref, w_ref, o_ref):
        x = jnp.maximum(e_ref[...] + b_ref[...], 0.0)
        o_ref[0] = jnp.dot(x, w_ref[0],
                           preferred_element_type=jnp.float32).astype(jnp.bfloat16)
    return pl.pallas_call(
        body,
        grid=(RTOT,),
        in_specs=[
            pl.BlockSpec((N, D), lambda r: (0, 0)),
            pl.BlockSpec((1, D), lambda r: (0, 0)),
            pl.BlockSpec((1, D, D), lambda r: (r, 0, 0)),
        ],
        out_specs=pl.BlockSpec((1, N, D), lambda r: (r, 0, 0)),
        out_shape=jax.ShapeDtypeStruct((RTOT, N, D), jnp.bfloat16),
    )(emb, bias, W)


def _transform(x, W):
    def body(x_ref, w_ref, o_ref):
        o_ref[0] = jnp.dot(x_ref[...], w_ref[0],
                           preferred_element_type=jnp.float32).astype(jnp.bfloat16)
    return pl.pallas_call(
        body,
        grid=(RTOT,),
        in_specs=[
            pl.BlockSpec((N, D), lambda r: (0, 0)),
            pl.BlockSpec((1, D, D), lambda r: (r, 0, 0)),
        ],
        out_specs=pl.BlockSpec((1, N, D), lambda r: (r, 0, 0)),
        out_shape=jax.ShapeDtypeStruct((RTOT, N, D), jnp.bfloat16),
    )(x, W)


def _combine(acc, yself, degp, b):
    def body(a_ref, ys_ref, dg_ref, b_ref, o_ref):
        ones = jnp.ones((NW, 1), jnp.float32)
        degsum = lax.dot_general(dg_ref[...], ones, (((0,), (0,)), ((), ())),
                                 preferred_element_type=jnp.float32)
        inv = 1.0 / (1.0 + degsum)
        tot = (a_ref[0].astype(jnp.float32) + a_ref[1].astype(jnp.float32)
               + ys_ref[...].astype(jnp.float32)) * inv + b_ref[...]
        o_ref[...] = jnp.maximum(tot, 0.0)
    return pl.pallas_call(
        body,
        out_shape=jax.ShapeDtypeStruct((N, D), jnp.float32),
    )(acc, yself, degp, b.reshape(1, D))


def _score(hs, ho, tp_col, relations):
    CH = 3072
    def body(hs_ref, ho_ref, tp_ref, rel_ref, s_ref, p_ref):
        rid = lax.broadcasted_iota(jnp.int32, (CH, NRELS), 1)
        onehot = (tp_ref[...] == rid).astype(jnp.float32)
        hr = jnp.dot(onehot, rel_ref[...], preferred_element_type=jnp.float32)
        prod = hs_ref[...] * hr * ho_ref[...]
        ones = jnp.ones((D, 1), jnp.float32)
        s_ref[...] = lax.dot_general(prod, ones, (((1,), (0,)), ((), ())),
                                     preferred_element_type=jnp.float32)
        p_ref[...] = jnp.sum(rel_ref[...] * rel_ref[...]).reshape(1, 1)
    return pl.pallas_call(
        body,
        grid=(NTP // CH,),
        in_specs=[
            pl.BlockSpec((CH, D), lambda i: (i, 0)),
            pl.BlockSpec((CH, D), lambda i: (i, 0)),
            pl.BlockSpec((CH, 1), lambda i: (i, 0)),
            pl.BlockSpec((NRELS, D), lambda i: (0, 0)),
        ],
        out_specs=[
            pl.BlockSpec((CH, 1), lambda i: (i, 0)),
            pl.BlockSpec((1, 1), lambda i: (0, 0)),
        ],
        out_shape=[
            jax.ShapeDtypeStruct((NTP, 1), jnp.float32),
            jax.ShapeDtypeStruct((1, 1), jnp.float32),
        ],
    )(hs, ho, tp_col, relations)


def _edge_agg(y_flat, eidx3, edst3):
    mesh = plsc.VectorSubcoreMesh(core_axis_name="c", subcore_axis_name="s")
    out_type = jax.ShapeDtypeStruct((NC, N, D), jnp.bfloat16)
    scratch = (
        [pltpu.VMEM_SHARED((N, D), jnp.bfloat16)]
        + [pltpu.VMEM((CHB, KB), jnp.int32)] * 2
        + [pltpu.VMEM((KB, D), jnp.bfloat16)] * RING
        + [pltpu.VMEM((ZROWS, D), jnp.bfloat16)]
        + [pltpu.SemaphoreType.DMA] * (2 * RING)
    )

    def body(y_hbm, ei_hbm, ed_hbm, acc_hbm, acc_sh, idx_v, dst_v, *rest):
        rbs = rest[:RING]
        zb = rest[RING]
        gsems = rest[RING + 1:2 * RING + 1]
        ssems = rest[2 * RING + 1:]
        c = lax.axis_index("c")
        s = lax.axis_index("s")
        wid = s * NC + c
        zvec = jnp.zeros((32,), jnp.bfloat16)

        def _zrow(i, cry):
            for j in range(D // 32):
                zb[i, pl.ds(j * 32, 32)] = zvec
            return cry
        lax.fori_loop(0, ZROWS, _zrow, 0)
        for k in range(RPT // ZROWS):
            pltpu.async_copy(zb, acc_sh.at[pl.ds(s * RPT + k * ZROWS, ZROWS)],
                             gsems[0])
        for k in range(RPT // ZROWS):
            pltpu.make_async_copy(zb, acc_sh.at[pl.ds(s * RPT, ZROWS)],
                                  gsems[0]).wait()
        plsc.subcore_barrier()

        def chunk(cc, cry):
            pltpu.sync_copy(ei_hbm.at[wid].at[pl.ds(cc * CHB, CHB)], idx_v)
            pltpu.sync_copy(ed_hbm.at[wid].at[pl.ds(cc * CHB, CHB)], dst_v)

            for u in range(RING):
                pltpu.async_copy(y_hbm.at[idx_v.at[u]], rbs[u], gsems[u])

            def stepr(t, cry2):
                base = RING * t
                for u in range(RING):
                    b = base + u
                    pltpu.make_async_copy(y_hbm.at[idx_v.at[b]], rbs[u],
                                          gsems[u]).wait()
                    pltpu.async_copy(rbs[u], acc_sh.at[dst_v.at[b]], ssems[u],
                                     add=True)
                for u in range(RING):
                    b = base + u
                    pltpu.make_async_copy(rbs[u], acc_sh.at[dst_v.at[b]],
                                          ssems[u]).wait()

                    @pl.when(b + RING < CHB)
                    def _():
                        pltpu.async_copy(y_hbm.at[idx_v.at[b + RING]], rbs[u],
                                         gsems[u])
                return cry2
            lax.fori_loop(0, CHB // RING, stepr, 0)
            return cry
        lax.fori_loop(0, NBLK // CHB, chunk, 0)

        plsc.subcore_barrier()
        pltpu.sync_copy(acc_sh.at[pl.ds(s * RPT, RPT)],
                        acc_hbm.at[c].at[pl.ds(s * RPT, RPT)])

    fn = pl.kernel(body, out_type=out_type, mesh=mesh,
                   scratch_types=scratch,
                   compiler_params=pltpu.CompilerParams(use_tc_tiling_on_sc=False, needs_layout_passes=False))
    return fn(y_flat, eidx3, edst3)


DCH = 4000


def _deg_hist(edst_flat):
    mesh = plsc.VectorSubcoreMesh(core_axis_name="c", subcore_axis_name="s")
    scratch = [
        pltpu.VMEM((DCH,), jnp.int32),
        pltpu.VMEM((N,), jnp.float32),
    ]

    def body(ed_hbm, deg_hbm, dst_v, deg_v):
        c = lax.axis_index("c")
        s = lax.axis_index("s")
        wid = s * NC + c
        zvec = jnp.zeros((16,), jnp.float32)
        ones16 = jnp.ones((16,), jnp.float32)

        def _zdeg(i, cry):
            deg_v[pl.ds(i * 16, 16)] = zvec
            return cry
        lax.fori_loop(0, N // 16, _zdeg, 0)

        def chunk(cc, cry):
            pltpu.sync_copy(ed_hbm.at[wid].at[pl.ds(cc * DCH, DCH)], dst_v)

            def dstep(j, cry2):
                idx16 = dst_v[pl.ds(j * 16, 16)]
                plsc.addupdate_scatter(deg_v, [idx16], ones16)
                return cry2
            lax.fori_loop(0, DCH // 16, dstep, 0)
            return cry
        lax.fori_loop(0, EPW // DCH, chunk, 0)
        pltpu.sync_copy(deg_v, deg_hbm.at[wid])

    fn = pl.kernel(body, out_type=jax.ShapeDtypeStruct((NW, N), jnp.float32),
                   mesh=mesh, scratch_types=scratch,
                   compiler_params=pltpu.CompilerParams(use_tc_tiling_on_sc=False, needs_layout_passes=False))
    return fn(edst_flat)


def _decoder_gather(x, tsp, top):
    mesh = plsc.VectorSubcoreMesh(core_axis_name="c", subcore_axis_name="s")
    out_type = tuple(jax.ShapeDtypeStruct((NTP, D), jnp.float32) for _ in range(2))
    scratch = [
        pltpu.VMEM((NBLK2, KB2), jnp.int32),
        pltpu.VMEM((NBLK2, KB2), jnp.int32),
        pltpu.VMEM((KB2, D), jnp.float32),
        pltpu.VMEM((KB2, D), jnp.float32),
        pltpu.VMEM((KB2, D), jnp.float32),
        pltpu.VMEM((KB2, D), jnp.float32),
        pltpu.SemaphoreType.DMA,
        pltpu.SemaphoreType.DMA,
        pltpu.SemaphoreType.DMA,
        pltpu.SemaphoreType.DMA,
        pltpu.SemaphoreType.DMA,
        pltpu.SemaphoreType.DMA,
        pltpu.SemaphoreType.DMA,
        pltpu.SemaphoreType.DMA,
    ]

    def body(x_hbm, ts_hbm, to_hbm, hs_hbm, ho_hbm, ts_v, to_v,
             rbs0, rbs1, rbo0, rbo1, gs0, gs1, go0, go1, ws0, ws1, wo0, wo1):
        c = lax.axis_index("c")
        s = lax.axis_index("s")
        wid = s * NC + c
        base = wid * (NBLK2 * KB2)
        pltpu.sync_copy(ts_hbm.at[wid], ts_v)
        pltpu.sync_copy(to_hbm.at[wid], to_v)

        pltpu.async_copy(x_hbm.at[ts_v.at[0]], rbs0, gs0)
        pltpu.async_copy(x_hbm.at[to_v.at[0]], rbo0, go0)
        pltpu.async_copy(x_hbm.at[ts_v.at[1]], rbs1, gs1)
        pltpu.async_copy(x_hbm.at[to_v.at[1]], rbo1, go1)

        def step(t, cry):
            b0 = 2 * t
            b1 = 2 * t + 1
            d0 = hs_hbm.at[pl.ds(base + b0 * KB2, KB2)]
            pltpu.make_async_copy(x_hbm.at[ts_v.at[b0]], rbs0, gs0).wait()
            pltpu.async_copy(rbs0, d0, ws0)
            e0 = ho_hbm.at[pl.ds(base + b0 * KB2, KB2)]
            pltpu.make_async_copy(x_hbm.at[to_v.at[b0]], rbo0, go0).wait()
            pltpu.async_copy(rbo0, e0, wo0)
            d1 = hs_hbm.at[pl.ds(base + b1 * KB2, KB2)]
            pltpu.make_async_copy(x_hbm.at[ts_v.at[b1]], rbs1, gs1).wait()
            pltpu.async_copy(rbs1, d1, ws1)
            e1 = ho_hbm.at[pl.ds(base + b1 * KB2, KB2)]
            pltpu.make_async_copy(x_hbm.at[to_v.at[b1]], rbo1, go1).wait()
            pltpu.async_copy(rbo1, e1, wo1)

            pltpu.make_async_copy(rbs0, d0, ws0).wait()
            pltpu.make_async_copy(rbo0, e0, wo0).wait()

            @pl.when(b1 + 1 < NBLK2)
            def _():
                pltpu.async_copy(x_hbm.at[ts_v.at[b1 + 1]], rbs0, gs0)
                pltpu.async_copy(x_hbm.at[to_v.at[b1 + 1]], rbo0, go0)
            pltpu.make_async_copy(rbs1, d1, ws1).wait()
            pltpu.make_async_copy(rbo1, e1, wo1).wait()

            @pl.when(b1 + 2 < NBLK2)
            def _():
                pltpu.async_copy(x_hbm.at[ts_v.at[b1 + 2]], rbs1, gs1)
                pltpu.async_copy(x_hbm.at[to_v.at[b1 + 2]], rbo1, go1)
            return cry
        lax.fori_loop(0, NBLK2 // 2, step, 0)

    fn = pl.kernel(body, out_type=out_type, mesh=mesh, scratch_types=scratch,
                   compiler_params=pltpu.CompilerParams(use_tc_tiling_on_sc=False, needs_layout_passes=False))
    return fn(x, tsp, top)


def kernel(graph, triples, node_embeddings, node_embeddings_bias, W1, b1, W2, b2, relations):
    s_ = graph[:, 0]
    p_ = graph[:, 1]
    o_ = graph[:, 2]
    eidx3 = jnp.concatenate([p_ * N + s_, (p_ + NRELS) * N + o_]).reshape(NW, NBLK, KB)
    edst = jnp.concatenate([o_, s_])
    edst3 = edst.reshape(NW, NBLK, KB)

    degp = _deg_hist(edst.reshape(NW, EPW))
    y1 = _transform1(node_embeddings, node_embeddings_bias, W1)
    acc1 = _edge_agg(y1.reshape(RTOT * N, D), eidx3, edst3)
    x1 = _combine(acc1, y1[2 * NRELS], degp, b1)

    y2 = _transform(x1, W2)
    acc2 = _edge_agg(y2.reshape(RTOT * N, D), eidx3, edst3)
    x2 = _combine(acc2, y2[2 * NRELS], degp, b2)

    tpad = jnp.zeros((NTP - NT,), jnp.int32)
    tsp = jnp.concatenate([triples[:, 0], tpad]).reshape(NW, NBLK2, KB2)
    top = jnp.concatenate([triples[:, 2], tpad]).reshape(NW, NBLK2, KB2)
    tp_col = jnp.concatenate([triples[:, 1], tpad]).reshape(NTP, 1)
    hs, ho = _decoder_gather(x2, tsp, top)
    scores_pad, pen = _score(hs, ho, tp_col, relations)

    return scores_pad.reshape(NTP)[:NT], pen[0, 0], x2


# speedup vs baseline: 1.5936x; 1.5936x over previous
import functools

import jax
import jax.numpy as jnp
from jax import lax
from jax.experimental import pallas as pl
from jax.experimental.pallas import tpu as pltpu
from jax.experimental.pallas import tpu_sc as plsc

N = 10000
NRELS = 18
RTOT = 2 * NRELS + 1
D = 128
NE = 320000
E2 = 2 * NE
NT = 30000

NC, NS = 2, 16
NW = NC * NS
EPW = E2 // NW
KB = 40
EPWD = EPW // 2
NBLK = EPWD // KB
RPT = N // NS
ZROWS = 25
CHB = 50
RING = 5

NTP = 30720
KB2 = 120
NBLK2 = (NTP // NW) // KB2


def _transform1(emb, bias, W):
    R = W.shape[0]
    def body(e_ref, b_ref, w_ref, o_ref):
        x = jnp.maximum(e_ref[...] + b_ref[...], 0.0)
        o_ref[0] = jnp.dot(x, w_ref[0], preferred_element_type=jnp.float32)
    return pl.pallas_call(
        body,
        grid=(R,),
        in_specs=[
            pl.BlockSpec((N, D), lambda r: (0, 0)),
            pl.BlockSpec((1, D), lambda r: (0, 0)),
            pl.BlockSpec((1, D, D), lambda r: (r, 0, 0)),
        ],
        out_specs=pl.BlockSpec((1, N, D), lambda r: (r, 0, 0)),
        out_shape=jax.ShapeDtypeStruct((R, N, D), jnp.float32),
    )(emb, bias, W)


def _transform(x, W):
    R = W.shape[0]
    def body(x_ref, w_ref, o_ref):
        o_ref[0] = jnp.dot(x_ref[...], w_ref[0], preferred_element_type=jnp.float32)
    return pl.pallas_call(
        body,
        grid=(R,),
        in_specs=[
            pl.BlockSpec((N, D), lambda r: (0, 0)),
            pl.BlockSpec((1, D, D), lambda r: (r, 0, 0)),
        ],
        out_specs=pl.BlockSpec((1, N, D), lambda r: (r, 0, 0)),
        out_shape=jax.ShapeDtypeStruct((R, N, D), jnp.float32),
    )(x, W)


def _combine(accf, acci, yself, degp, b):
    def body(af_ref, ai_ref, ys_ref, dg_ref, b_ref, o_ref):
        ones = jnp.ones((NW, 1), jnp.float32)
        degsum = lax.dot_general(dg_ref[...], ones, (((0,), (0,)), ((), ())),
                                 preferred_element_type=jnp.float32)
        inv = 1.0 / (1.0 + degsum)
        tot = (af_ref[0] + af_ref[1] + ai_ref[0] + ai_ref[1]
               + ys_ref[...]) * inv + b_ref[...]
        o_ref[...] = jnp.maximum(tot, 0.0)
    return pl.pallas_call(
        body,
        out_shape=jax.ShapeDtypeStruct((N, D), jnp.float32),
    )(accf, acci, yself, degp, b.reshape(1, D))


def _score(hs, ho, tp_col, relations):
    CH = 3072
    def body(hs_ref, ho_ref, tp_ref, rel_ref, s_ref, p_ref):
        rid = lax.broadcasted_iota(jnp.int32, (CH, NRELS), 1)
        onehot = (tp_ref[...] == rid).astype(jnp.float32)
        hr = jnp.dot(onehot, rel_ref[...], preferred_element_type=jnp.float32)
        prod = hs_ref[...] * hr * ho_ref[...]
        ones = jnp.ones((D, 1), jnp.float32)
        s_ref[...] = lax.dot_general(prod, ones, (((1,), (0,)), ((), ())),
                                     preferred_element_type=jnp.float32)
        p_ref[...] = jnp.sum(rel_ref[...] * rel_ref[...]).reshape(1, 1)
    return pl.pallas_call(
        body,
        grid=(NTP // CH,),
        in_specs=[
            pl.BlockSpec((CH, D), lambda i: (i, 0)),
            pl.BlockSpec((CH, D), lambda i: (i, 0)),
            pl.BlockSpec((CH, 1), lambda i: (i, 0)),
            pl.BlockSpec((NRELS, D), lambda i: (0, 0)),
        ],
        out_specs=[
            pl.BlockSpec((CH, 1), lambda i: (i, 0)),
            pl.BlockSpec((1, 1), lambda i: (0, 0)),
        ],
        out_shape=[
            jax.ShapeDtypeStruct((NTP, 1), jnp.float32),
            jax.ShapeDtypeStruct((1, 1), jnp.float32),
        ],
    )(hs, ho, tp_col, relations)


def _edge_agg(y_flat, eidx3, edst3):
    mesh = plsc.VectorSubcoreMesh(core_axis_name="c", subcore_axis_name="s")
    out_type = jax.ShapeDtypeStruct((NC, N, D), jnp.float32)
    scratch = (
        [pltpu.VMEM_SHARED((N, D), jnp.float32)]
        + [pltpu.VMEM((CHB, KB), jnp.int32)] * 2
        + [pltpu.VMEM((KB, D), jnp.float32)] * RING
        + [pltpu.VMEM((ZROWS, D), jnp.float32)]
        + [pltpu.SemaphoreType.DMA] * (2 * RING)
    )

    def body(y_hbm, ei_hbm, ed_hbm, acc_hbm, acc_sh, idx_v, dst_v, *rest):
        rbs = rest[:RING]
        zb = rest[RING]
        gsems = rest[RING + 1:2 * RING + 1]
        ssems = rest[2 * RING + 1:]
        c = lax.axis_index("c")
        s = lax.axis_index("s")
        wid = s * NC + c
        zvec = jnp.zeros((16,), jnp.float32)

        def _zrow(i, cry):
            for j in range(D // 16):
                zb[i, pl.ds(j * 16, 16)] = zvec
            return cry
        lax.fori_loop(0, ZROWS, _zrow, 0)
        for k in range(RPT // ZROWS):
            pltpu.async_copy(zb, acc_sh.at[pl.ds(s * RPT + k * ZROWS, ZROWS)],
                             gsems[0])
        for k in range(RPT // ZROWS):
            pltpu.make_async_copy(zb, acc_sh.at[pl.ds(s * RPT, ZROWS)],
                                  gsems[0]).wait()
        plsc.subcore_barrier()

        def chunk(cc, cry):
            pltpu.sync_copy(ei_hbm.at[wid].at[pl.ds(cc * CHB, CHB)], idx_v)
            pltpu.sync_copy(ed_hbm.at[wid].at[pl.ds(cc * CHB, CHB)], dst_v)

            for u in range(RING):
                pltpu.async_copy(y_hbm.at[idx_v.at[u]], rbs[u], gsems[u])

            def stepr(t, cry2):
                base = RING * t
                for u in range(RING):
                    b = base + u
                    pltpu.make_async_copy(y_hbm.at[idx_v.at[b]], rbs[u],
                                          gsems[u]).wait()
                    pltpu.async_copy(rbs[u], acc_sh.at[dst_v.at[b]], ssems[u],
                                     add=True)
                for u in range(RING):
                    b = base + u
                    pltpu.make_async_copy(rbs[u], acc_sh.at[dst_v.at[b]],
                                          ssems[u]).wait()

                    @pl.when(b + RING < CHB)
                    def _():
                        pltpu.async_copy(y_hbm.at[idx_v.at[b + RING]], rbs[u],
                                         gsems[u])
                return cry2
            lax.fori_loop(0, CHB // RING, stepr, 0)
            return cry
        lax.fori_loop(0, NBLK // CHB, chunk, 0)

        plsc.subcore_barrier()
        pltpu.sync_copy(acc_sh.at[pl.ds(s * RPT, RPT)],
                        acc_hbm.at[c].at[pl.ds(s * RPT, RPT)])

    fn = pl.kernel(body, out_type=out_type, mesh=mesh,
                   scratch_types=scratch,
                   compiler_params=pltpu.CompilerParams(use_tc_tiling_on_sc=False, needs_layout_passes=False))
    return fn(y_flat, eidx3, edst3)


DCH = 4000


def _deg_hist(edst_flat):
    mesh = plsc.VectorSubcoreMesh(core_axis_name="c", subcore_axis_name="s")
    scratch = [
        pltpu.VMEM((DCH,), jnp.int32),
        pltpu.VMEM((N,), jnp.float32),
    ]

    def body(ed_hbm, deg_hbm, dst_v, deg_v):
        c = lax.axis_index("c")
        s = lax.axis_index("s")
        wid = s * NC + c
        zvec = jnp.zeros((16,), jnp.float32)
        ones16 = jnp.ones((16,), jnp.float32)

        def _zdeg(i, cry):
            deg_v[pl.ds(i * 16, 16)] = zvec
            return cry
        lax.fori_loop(0, N // 16, _zdeg, 0)

        def chunk(cc, cry):
            pltpu.sync_copy(ed_hbm.at[wid].at[pl.ds(cc * DCH, DCH)], dst_v)

            def dstep(j, cry2):
                idx16 = dst_v[pl.ds(j * 16, 16)]
                plsc.addupdate_scatter(deg_v, [idx16], ones16)
                return cry2
            lax.fori_loop(0, DCH // 16, dstep, 0)
            return cry
        lax.fori_loop(0, EPW // DCH, chunk, 0)
        pltpu.sync_copy(deg_v, deg_hbm.at[wid])

    fn = pl.kernel(body, out_type=jax.ShapeDtypeStruct((NW, N), jnp.float32),
                   mesh=mesh, scratch_types=scratch,
                   compiler_params=pltpu.CompilerParams(use_tc_tiling_on_sc=False, needs_layout_passes=False))
    return fn(edst_flat)


def _decoder_gather(x, tsp, top):
    mesh = plsc.VectorSubcoreMesh(core_axis_name="c", subcore_axis_name="s")
    out_type = tuple(jax.ShapeDtypeStruct((NTP, D), jnp.float32) for _ in range(2))
    scratch = [
        pltpu.VMEM((NBLK2, KB2), jnp.int32),
        pltpu.VMEM((NBLK2, KB2), jnp.int32),
        pltpu.VMEM((KB2, D), jnp.float32),
        pltpu.VMEM((KB2, D), jnp.float32),
        pltpu.VMEM((KB2, D), jnp.float32),
        pltpu.VMEM((KB2, D), jnp.float32),
        pltpu.SemaphoreType.DMA,
        pltpu.SemaphoreType.DMA,
        pltpu.SemaphoreType.DMA,
        pltpu.SemaphoreType.DMA,
        pltpu.SemaphoreType.DMA,
        pltpu.SemaphoreType.DMA,
        pltpu.SemaphoreType.DMA,
        pltpu.SemaphoreType.DMA,
    ]

    def body(x_hbm, ts_hbm, to_hbm, hs_hbm, ho_hbm, ts_v, to_v,
             rbs0, rbs1, rbo0, rbo1, gs0, gs1, go0, go1, ws0, ws1, wo0, wo1):
        c = lax.axis_index("c")
        s = lax.axis_index("s")
        wid = s * NC + c
        base = wid * (NBLK2 * KB2)
        pltpu.sync_copy(ts_hbm.at[wid], ts_v)
        pltpu.sync_copy(to_hbm.at[wid], to_v)

        pltpu.async_copy(x_hbm.at[ts_v.at[0]], rbs0, gs0)
        pltpu.async_copy(x_hbm.at[to_v.at[0]], rbo0, go0)
        pltpu.async_copy(x_hbm.at[ts_v.at[1]], rbs1, gs1)
        pltpu.async_copy(x_hbm.at[to_v.at[1]], rbo1, go1)

        def step(t, cry):
            b0 = 2 * t
            b1 = 2 * t + 1
            d0 = hs_hbm.at[pl.ds(base + b0 * KB2, KB2)]
            pltpu.make_async_copy(x_hbm.at[ts_v.at[b0]], rbs0, gs0).wait()
            pltpu.async_copy(rbs0, d0, ws0)
            e0 = ho_hbm.at[pl.ds(base + b0 * KB2, KB2)]
            pltpu.make_async_copy(x_hbm.at[to_v.at[b0]], rbo0, go0).wait()
            pltpu.async_copy(rbo0, e0, wo0)
            d1 = hs_hbm.at[pl.ds(base + b1 * KB2, KB2)]
            pltpu.make_async_copy(x_hbm.at[ts_v.at[b1]], rbs1, gs1).wait()
            pltpu.async_copy(rbs1, d1, ws1)
            e1 = ho_hbm.at[pl.ds(base + b1 * KB2, KB2)]
            pltpu.make_async_copy(x_hbm.at[to_v.at[b1]], rbo1, go1).wait()
            pltpu.async_copy(rbo1, e1, wo1)

            pltpu.make_async_copy(rbs0, d0, ws0).wait()
            pltpu.make_async_copy(rbo0, e0, wo0).wait()

            @pl.when(b1 + 1 < NBLK2)
            def _():
                pltpu.async_copy(x_hbm.at[ts_v.at[b1 + 1]], rbs0, gs0)
                pltpu.async_copy(x_hbm.at[to_v.at[b1 + 1]], rbo0, go0)
            pltpu.make_async_copy(rbs1, d1, ws1).wait()
            pltpu.make_async_copy(rbo1, e1, wo1).wait()

            @pl.when(b1 + 2 < NBLK2)
            def _():
                pltpu.async_copy(x_hbm.at[ts_v.at[b1 + 2]], rbs1, gs1)
                pltpu.async_copy(x_hbm.at[to_v.at[b1 + 2]], rbo1, go1)
            return cry
        lax.fori_loop(0, NBLK2 // 2, step, 0)

    fn = pl.kernel(body, out_type=out_type, mesh=mesh, scratch_types=scratch,
                   compiler_params=pltpu.CompilerParams(use_tc_tiling_on_sc=False, needs_layout_passes=False))
    return fn(x, tsp, top)


def kernel(graph, triples, node_embeddings, node_embeddings_bias, W1, b1, W2, b2, relations):
    s_ = graph[:, 0]
    p_ = graph[:, 1]
    o_ = graph[:, 2]
    eif = (p_ * N + s_).reshape(NW, NBLK, KB)
    edf = o_.reshape(NW, NBLK, KB)
    eii = (p_ * N + o_).reshape(NW, NBLK, KB)
    edi = s_.reshape(NW, NBLK, KB)
    edall = jnp.concatenate([o_, s_]).reshape(NW, EPW)

    degp = _deg_hist(edall)
    y1f = _transform1(node_embeddings, node_embeddings_bias, W1[:NRELS])
    acc1f = _edge_agg(y1f.reshape(NRELS * N, D), eif, edf)
    y1i = _transform1(node_embeddings, node_embeddings_bias, W1[NRELS:])
    acc1i = _edge_agg(y1i.reshape((NRELS + 1) * N, D), eii, edi)
    x1 = _combine(acc1f, acc1i, y1i[NRELS], degp, b1)

    y2f = _transform(x1, W2[:NRELS])
    acc2f = _edge_agg(y2f.reshape(NRELS * N, D), eif, edf)
    y2i = _transform(x1, W2[NRELS:])
    acc2i = _edge_agg(y2i.reshape((NRELS + 1) * N, D), eii, edi)
    x2 = _combine(acc2f, acc2i, y2i[NRELS], degp, b2)

    tpad = jnp.zeros((NTP - NT,), jnp.int32)
    tsp = jnp.concatenate([triples[:, 0], tpad]).reshape(NW, NBLK2, KB2)
    top = jnp.concatenate([triples[:, 2], tpad]).reshape(NW, NBLK2, KB2)
    tp_col = jnp.concatenate([triples[:, 1], tpad]).reshape(NTP, 1)
    hs, ho = _decoder_gather(x2, tsp, top)
    scores_pad, pen = _score(hs, ho, tp_col, relations)

    return scores_pad.reshape(NTP)[:NT], pen[0, 0], x2
